# baseline (device time: 127485 ns/iter reference)
import jax
import jax.numpy as jnp
from jax import lax
from jax.experimental import pallas as pl
from jax.experimental.pallas import tpu as pltpu

N_DEV = 8
_GELU_C = 0.7978845608028654

_PIECES = ((0, 176), (176, 176), (352, 160))
NSUB = 4


def _split(r0, nr, n):
    per = (nr // n // 16) * 16
    sizes = [per] * (n - 1) + [nr - per * (n - 1)]
    out, off = [], r0
    for s in sizes:
        out.append((off, s))
        off += s
    return tuple(out)


_SUBS = tuple(_split(r0, nr, NSUB) for r0, nr in _PIECES)


def _lab(p):
    low = p & 3
    return (p & 4) | (low ^ (low >> 1))


def kernel(x, w_mat):
    m_per, k = x.shape
    _, n_per = w_mat.shape

    def body(x_ref, w_ref, out_ref, chunk_buf, send_sems, recv_sems):
        my = lax.axis_index("i")
        my_lab = _lab(my)

        barrier_sem = pltpu.get_barrier_semaphore()
        for dim in range(3):
            pl.semaphore_signal(
                barrier_sem, inc=1,
                device_id=(_lab(my_lab ^ (1 << dim)),),
                device_id_type=pl.DeviceIdType.MESH,
            )
        pl.semaphore_wait(barrier_sem, 3)

        sends = []

        def send_sub(j, r, dim, u):
            slot = _lab(my_lab ^ r)
            nb = _lab(my_lab ^ (1 << dim))
            r0, nr = _SUBS[j][u]
            rd = pltpu.make_async_remote_copy(
                src_ref=chunk_buf.at[slot, pl.ds(r0, nr)],
                dst_ref=chunk_buf.at[slot, pl.ds(r0, nr)],
                send_sem=send_sems.at[len(sends)],
                recv_sem=recv_sems.at[j, r ^ (1 << dim), u],
                device_id=(nb,),
                device_id_type=pl.DeviceIdType.MESH,
            )
            rd.start()
            sends.append(rd)

        def wait_sub(j, r, u):
            slot = _lab(my_lab ^ r)
            r0, nr = _SUBS[j][u]
            rd = pltpu.make_async_remote_copy(
                src_ref=chunk_buf.at[slot, pl.ds(r0, nr)],
                dst_ref=chunk_buf.at[slot, pl.ds(r0, nr)],
                send_sem=send_sems.at[0],
                recv_sem=recv_sems.at[j, r, u],
                device_id=(my,),
                device_id_type=pl.DeviceIdType.MESH,
            )
            rd.wait_recv()

        for u in range(NSUB):
            for j in range(3):
                r0, nr = _SUBS[j][u]
                chunk_buf[my, pl.ds(r0, nr)] = \
                    x_ref[r0:r0 + nr, :].astype(jnp.bfloat16)
                send_sub(j, 0, j, u)

        w_bf = w_ref[...].astype(jnp.bfloat16)

        def compute_piece(j, r):
            slot = _lab(my_lab ^ r)
            r0, nr = _PIECES[j]
            y = jnp.dot(chunk_buf[slot, pl.ds(r0, nr)], w_bf,
                        preferred_element_type=jnp.float32)
            y = 0.5 * y * (1.0 + jnp.tanh(_GELU_C * (y + 0.044715 * y * y * y)))
            out_ref[pl.ds(slot * m_per + r0, nr), :] = y

        for j in range(3):
            compute_piece(j, 0)

        for u in range(NSUB):
            for j in range(3):
                j1, j2 = (j + 1) % 3, (j + 2) % 3
                bj = 1 << j
                wait_sub(j, bj, u)
                send_sub(j, bj, j1, u)
                send_sub(j, bj, j2, u)
        for j in range(3):
            compute_piece(j, 1 << j)
        for u in range(NSUB):
            for j in range(3):
                j1, j2 = (j + 1) % 3, (j + 2) % 3
                bj, b1, b2 = 1 << j, 1 << j1, 1 << j2
                wait_sub(j, bj | b1, u)
                send_sub(j, bj | b1, j2, u)
                send_sub(j, bj | b1, j, u)
                wait_sub(j, bj | b2, u)
                send_sub(j, bj | b2, j, u)
        for j in range(3):
            j1, j2 = (j + 1) % 3, (j + 2) % 3
            bj = 1 << j
            compute_piece(j, bj | (1 << j1))
            compute_piece(j, bj | (1 << j2))
        for u in range(NSUB):
            for j in range(3):
                wait_sub(j, 7, u)
                send_sub(j, 7, j, u)
        for u in range(NSUB):
            for j in range(3):
                j1, j2 = (j + 1) % 3, (j + 2) % 3
                wait_sub(j, 1 << j1, u)
                wait_sub(j, 1 << j2, u)
        for j in range(3):
            j1, j2 = (j + 1) % 3, (j + 2) % 3
            compute_piece(j, 7)
            compute_piece(j, 1 << j1)
            compute_piece(j, 1 << j2)
        for j in range(3):
            j1, j2 = (j + 1) % 3, (j + 2) % 3
            for u in range(NSUB):
                wait_sub(j, (1 << j1) | (1 << j2), u)
            compute_piece(j, (1 << j1) | (1 << j2))
        for rd in sends:
            rd.wait_send()

    out_shape = jax.ShapeDtypeStruct((N_DEV * m_per, n_per), jnp.float32)
    return pl.pallas_call(
        body,
        out_shape=out_shape,
        in_specs=[
            pl.BlockSpec(memory_space=pltpu.VMEM),
            pl.BlockSpec(memory_space=pltpu.VMEM),
        ],
        out_specs=pl.BlockSpec(memory_space=pltpu.VMEM),
        scratch_shapes=[
            pltpu.VMEM((N_DEV, m_per, k), jnp.bfloat16),
            pltpu.SemaphoreType.DMA((21 * NSUB,)),
            pltpu.SemaphoreType.DMA((3, 8, NSUB)),
        ],
        compiler_params=pltpu.CompilerParams(
            collective_id=0,
            vmem_limit_bytes=100 * 1024 * 1024,
        ),
    )(x, w_mat)


# device time: 126626 ns/iter; 1.0068x vs baseline; 1.0068x over previous
import jax
import jax.numpy as jnp
from jax import lax
from jax.experimental import pallas as pl
from jax.experimental.pallas import tpu as pltpu

N_DEV = 8
_GELU_C = 0.7978845608028654

_PIECES = ((0, 176), (176, 176), (352, 160))
NSUB = 2


def _split(r0, nr, n):
    per = (nr // n // 16) * 16
    sizes = [per] * (n - 1) + [nr - per * (n - 1)]
    out, off = [], r0
    for s in sizes:
        out.append((off, s))
        off += s
    return tuple(out)


_SUBS = tuple(_split(r0, nr, NSUB) for r0, nr in _PIECES)


def _lab(p):
    low = p & 3
    return (p & 4) | (low ^ (low >> 1))


def kernel(x, w_mat):
    m_per, k = x.shape
    _, n_per = w_mat.shape

    def body(x_ref, w_ref, out_ref, chunk_buf, send_sems, recv_sems):
        my = lax.axis_index("i")
        my_lab = _lab(my)

        barrier_sem = pltpu.get_barrier_semaphore()
        for dim in range(3):
            pl.semaphore_signal(
                barrier_sem, inc=1,
                device_id=(_lab(my_lab ^ (1 << dim)),),
                device_id_type=pl.DeviceIdType.MESH,
            )
        pl.semaphore_wait(barrier_sem, 3)

        sends = []

        def send_sub(j, r, dim, u):
            slot = _lab(my_lab ^ r)
            nb = _lab(my_lab ^ (1 << dim))
            r0, nr = _SUBS[j][u]
            rd = pltpu.make_async_remote_copy(
                src_ref=chunk_buf.at[slot, pl.ds(r0, nr)],
                dst_ref=chunk_buf.at[slot, pl.ds(r0, nr)],
                send_sem=send_sems.at[len(sends)],
                recv_sem=recv_sems.at[j, r ^ (1 << dim), u],
                device_id=(nb,),
                device_id_type=pl.DeviceIdType.MESH,
            )
            rd.start()
            sends.append(rd)

        def wait_sub(j, r, u):
            slot = _lab(my_lab ^ r)
            r0, nr = _SUBS[j][u]
            rd = pltpu.make_async_remote_copy(
                src_ref=chunk_buf.at[slot, pl.ds(r0, nr)],
                dst_ref=chunk_buf.at[slot, pl.ds(r0, nr)],
                send_sem=send_sems.at[0],
                recv_sem=recv_sems.at[j, r, u],
                device_id=(my,),
                device_id_type=pl.DeviceIdType.MESH,
            )
            rd.wait_recv()

        for u in range(NSUB):
            for j in range(3):
                r0, nr = _SUBS[j][u]
                chunk_buf[my, pl.ds(r0, nr)] = \
                    x_ref[r0:r0 + nr, :].astype(jnp.bfloat16)
                send_sub(j, 0, j, u)

        w_bf = w_ref[...].astype(jnp.bfloat16)

        def compute_piece(j, r):
            slot = _lab(my_lab ^ r)
            r0, nr = _PIECES[j]
            y = jnp.dot(chunk_buf[slot, pl.ds(r0, nr)], w_bf,
                        preferred_element_type=jnp.float32)
            y = 0.5 * y * (1.0 + jnp.tanh(_GELU_C * (y + 0.044715 * y * y * y)))
            out_ref[pl.ds(slot * m_per + r0, nr), :] = y

        for j in range(3):
            compute_piece(j, 0)

        for u in range(NSUB):
            for j in range(3):
                j1, j2 = (j + 1) % 3, (j + 2) % 3
                bj = 1 << j
                wait_sub(j, bj, u)
                send_sub(j, bj, j1, u)
                send_sub(j, bj, j2, u)
        for j in range(3):
            compute_piece(j, 1 << j)
        for u in range(NSUB):
            for j in range(3):
                j1, j2 = (j + 1) % 3, (j + 2) % 3
                bj, b1, b2 = 1 << j, 1 << j1, 1 << j2
                wait_sub(j, bj | b1, u)
                send_sub(j, bj | b1, j2, u)
                send_sub(j, bj | b1, j, u)
                wait_sub(j, bj | b2, u)
                send_sub(j, bj | b2, j, u)
        for j in range(3):
            j1, j2 = (j + 1) % 3, (j + 2) % 3
            bj = 1 << j
            compute_piece(j, bj | (1 << j1))
            compute_piece(j, bj | (1 << j2))
        for u in range(NSUB):
            for j in range(3):
                wait_sub(j, 7, u)
                send_sub(j, 7, j, u)
        for u in range(NSUB):
            for j in range(3):
                j1, j2 = (j + 1) % 3, (j + 2) % 3
                wait_sub(j, 1 << j1, u)
                wait_sub(j, 1 << j2, u)
        for j in range(3):
            j1, j2 = (j + 1) % 3, (j + 2) % 3
            compute_piece(j, 7)
            compute_piece(j, 1 << j1)
            compute_piece(j, 1 << j2)
        for j in range(3):
            j1, j2 = (j + 1) % 3, (j + 2) % 3
            for u in range(NSUB):
                wait_sub(j, (1 << j1) | (1 << j2), u)
            compute_piece(j, (1 << j1) | (1 << j2))
        for rd in sends:
            rd.wait_send()

    out_shape = jax.ShapeDtypeStruct((N_DEV * m_per, n_per), jnp.float32)
    return pl.pallas_call(
        body,
        out_shape=out_shape,
        in_specs=[
            pl.BlockSpec(memory_space=pltpu.VMEM),
            pl.BlockSpec(memory_space=pltpu.VMEM),
        ],
        out_specs=pl.BlockSpec(memory_space=pltpu.VMEM),
        scratch_shapes=[
            pltpu.VMEM((N_DEV, m_per, k), jnp.bfloat16),
            pltpu.SemaphoreType.DMA((21 * NSUB,)),
            pltpu.SemaphoreType.DMA((3, 8, NSUB)),
        ],
        compiler_params=pltpu.CompilerParams(
            collective_id=0,
            vmem_limit_bytes=100 * 1024 * 1024,
        ),
    )(x, w_mat)
